# baseline (speedup 1.0000x reference)
import functools

import jax
import jax.numpy as jnp
from jax import lax
from jax.experimental import pallas as pl
from jax.experimental.pallas import tpu as pltpu
from jax.experimental.pallas import tpu_sc as plsc

N_FIELDS = 26
VOCAB = 100000
DIM = 32
BATCH = 16384

NC, NS = 2, 16
NW = NC * NS
BW = BATCH // NW
CB = 32
NCHUNK = BW // CB
ROWS = CB * N_FIELDS
GCHUNK = 104
NG = ROWS // GCHUNK
NV = ROWS // 16


def kernel(indices, tables, weights):
    flat_tables = tables.reshape(N_FIELDS * (VOCAB + 1), DIM)
    flat_idx = indices.reshape(BATCH * N_FIELDS)
    wrow = jnp.broadcast_to(jnp.tile(weights, CB)[:, None], (ROWS, DIM))
    mesh = plsc.VectorSubcoreMesh(core_axis_name="c", subcore_axis_name="s")

    @functools.partial(
        pl.kernel,
        out_type=jax.ShapeDtypeStruct((BATCH * N_FIELDS, DIM), jnp.float32),
        mesh=mesh,
        compiler_params=pltpu.CompilerParams(use_tc_tiling_on_sc=False),
        scratch_types=[
            pltpu.VMEM((ROWS,), jnp.int32),
            pltpu.VMEM((ROWS, DIM), jnp.float32),
            pltpu.VMEM((ROWS,), jnp.int32),
            pltpu.VMEM((ROWS, DIM), jnp.float32),
            pltpu.SemaphoreType.DMA,
        ],
    )
    def fk(idx_hbm, tbl_hbm, w_hbm, out_hbm, offs_v, wrow_v, g_v,
           rows_v, sem):
        wid = lax.axis_index("s") * NC + lax.axis_index("c")
        pltpu.sync_copy(w_hbm, wrow_v)
        lane = lax.iota(jnp.int32, 16)

        for k in range(NV):
            f16 = (lane + k * 16) % N_FIELDS
            offs_v[pl.ds(k * 16, 16)] = f16 * (VOCAB + 1)

        def chunk_body(c, carry):
            start = pl.multiple_of(wid * (BW * N_FIELDS) + c * ROWS, ROWS)
            pltpu.sync_copy(idx_hbm.at[pl.ds(start, ROWS)], g_v)
            for k in range(NV):
                raw = g_v[pl.ds(k * 16, 16)]
                ok = (raw >= 0) & (raw < VOCAB)
                g_v[pl.ds(k * 16, 16)] = (
                    jnp.where(ok, raw + 1, 0) + offs_v[pl.ds(k * 16, 16)]
                )
            for j in range(NG):
                pltpu.async_copy(
                    tbl_hbm.at[g_v.at[pl.ds(j * GCHUNK, GCHUNK)]],
                    rows_v.at[pl.ds(j * GCHUNK, GCHUNK)],
                    sem,
                )
            for j in range(NG):
                pltpu.make_async_copy(
                    tbl_hbm.at[g_v.at[pl.ds(j * GCHUNK, GCHUNK)]],
                    rows_v.at[pl.ds(j * GCHUNK, GCHUNK)],
                    sem,
                ).wait()

            def mul_body(r, carry2):
                rows_v[r, pl.ds(0, 16)] = (
                    rows_v[r, pl.ds(0, 16)] * wrow_v[r, pl.ds(0, 16)]
                )
                rows_v[r, pl.ds(16, 16)] = (
                    rows_v[r, pl.ds(16, 16)] * wrow_v[r, pl.ds(16, 16)]
                )
                return carry2

            lax.fori_loop(0, ROWS, mul_body, 0, unroll=8)
            pltpu.sync_copy(rows_v, out_hbm.at[pl.ds(start, ROWS), :])
            return carry

        lax.fori_loop(0, NCHUNK, chunk_body, 0)

    out = fk(flat_idx, flat_tables, wrow)
    return out.reshape(BATCH, N_FIELDS * DIM)


# speedup vs baseline: 2.4926x; 2.4926x over previous
import functools

import jax
import jax.numpy as jnp
from jax import lax
from jax.experimental import pallas as pl
from jax.experimental.pallas import tpu as pltpu
from jax.experimental.pallas import tpu_sc as plsc

N_FIELDS = 26
VOCAB = 100000
DIM = 32
BATCH = 16384

NC, NS = 2, 16
NW = NC * NS
BW = BATCH // NW
GCHUNK = 128
NG = BW // GCHUNK
NV = BW // 16


def kernel(indices, tables, weights):
    idx_t = indices.T
    wb = jnp.broadcast_to(weights[:, None], (N_FIELDS, 16))
    mesh = plsc.VectorSubcoreMesh(core_axis_name="c", subcore_axis_name="s")

    @functools.partial(
        pl.kernel,
        out_type=jax.ShapeDtypeStruct((BATCH, N_FIELDS * DIM), jnp.float32),
        mesh=mesh,
        compiler_params=pltpu.CompilerParams(use_tc_tiling_on_sc=False),
        scratch_types=[
            pltpu.VMEM((N_FIELDS, 16), jnp.float32),
            pltpu.VMEM((BW,), jnp.int32),
            pltpu.VMEM((BW, DIM), jnp.float32),
            pltpu.SemaphoreType.DMA,
        ],
    )
    def fk(idx_hbm, tbl_hbm, w_hbm, out_hbm, w_v, g_v, rows_v, sem):
        wid = lax.axis_index("s") * NC + lax.axis_index("c")
        b0 = pl.multiple_of(wid * BW, BW)
        pltpu.sync_copy(w_hbm, w_v)

        def field_body(f, carry):
            pltpu.sync_copy(idx_hbm.at[f, pl.ds(b0, BW)], g_v)
            for k in range(NV):
                raw = g_v[pl.ds(k * 16, 16)]
                ok = (raw >= 0) & (raw < VOCAB)
                g_v[pl.ds(k * 16, 16)] = jnp.where(ok, raw + 1, 0)
            for j in range(NG):
                pltpu.async_copy(
                    tbl_hbm.at[f].at[g_v.at[pl.ds(j * GCHUNK, GCHUNK)]],
                    rows_v.at[pl.ds(j * GCHUNK, GCHUNK)],
                    sem,
                )
            for j in range(NG):
                pltpu.make_async_copy(
                    tbl_hbm.at[f].at[g_v.at[pl.ds(j * GCHUNK, GCHUNK)]],
                    rows_v.at[pl.ds(j * GCHUNK, GCHUNK)],
                    sem,
                ).wait()
            wvec = w_v[f]

            def mul_body(r, carry2):
                rows_v[r, pl.ds(0, 16)] = rows_v[r, pl.ds(0, 16)] * wvec
                rows_v[r, pl.ds(16, 16)] = rows_v[r, pl.ds(16, 16)] * wvec
                return carry2

            lax.fori_loop(0, BW, mul_body, 0, unroll=8)
            pltpu.sync_copy(
                rows_v, out_hbm.at[pl.ds(b0, BW), pl.ds(f * DIM, DIM)]
            )
            return carry

        lax.fori_loop(0, N_FIELDS, field_body, 0)

    return fk(idx_t, tables, wb)
